# baseline (speedup 1.0000x reference)
import functools

import jax
import jax.numpy as jnp
from jax import lax
from jax.experimental import pallas as pl
from jax.experimental.pallas import tpu as pltpu
from jax.experimental.pallas import tpu_sc as plsc

N = 10000
E = 320000
FEAT = 128
FH = FEAT // 2
H = 8
HD = FEAT // H
HH = H // 2
NC = 2
NS = 16
NW = NC * NS
EPW = E // NW
C = 80
G = C // 16
NCHUNK = EPW // C
NPAD = 10240
RPT = NPAD // NS
PW = 80
CHT = NW * NCHUNK
ERP = CHT * 8


def _proj_body(x_ref, wq_ref, wk_ref, wv_ref, q_ref, k_ref, v_ref):
    x = x_ref[...]
    q_ref[...] = x @ wq_ref[...]
    k_ref[...] = x @ wk_ref[...]
    v_ref[...] = x @ wv_ref[...]


def _project(nfeat, WqT, WkT, WvT):
    blk = 2000
    return pl.pallas_call(
        _proj_body,
        grid=(N // blk,),
        in_specs=[
            pl.BlockSpec((blk, FEAT), lambda i: (i, 0)),
            pl.BlockSpec((FEAT, FEAT), lambda i: (0, 0)),
            pl.BlockSpec((FEAT, FEAT), lambda i: (0, 0)),
            pl.BlockSpec((FEAT, FEAT), lambda i: (0, 0)),
        ],
        out_specs=[
            pl.BlockSpec((blk, FEAT), lambda i: (i, 0)),
            pl.BlockSpec((blk, FEAT), lambda i: (i, 0)),
            pl.BlockSpec((blk, FEAT), lambda i: (i, 0)),
        ],
        out_shape=[jax.ShapeDtypeStruct((N, FEAT), jnp.float32)] * 3,
    )(nfeat, WqT, WkT, WvT)


_MESH = plsc.VectorSubcoreMesh(core_axis_name="c", subcore_axis_name="s")


@functools.partial(
    pl.kernel,
    mesh=_MESH,
    compiler_params=pltpu.CompilerParams(needs_layout_passes=False),
    out_type=[
        jax.ShapeDtypeStruct((NC, NPAD, PW), jnp.float32),
        jax.ShapeDtypeStruct((NC, NPAD, PW), jnp.float32),
        jax.ShapeDtypeStruct((ERP, 128), jnp.float32),
    ],
    scratch_types=[
        pltpu.VMEM((C,), jnp.int32),
        pltpu.VMEM((C,), jnp.int32),
        pltpu.VMEM((C, FEAT), jnp.float32),
        pltpu.VMEM((C, FEAT), jnp.float32),
        pltpu.VMEM((C, FEAT), jnp.float32),
        pltpu.VMEM((C, PW), jnp.float32),
        pltpu.VMEM((8, 128), jnp.float32),
        pltpu.VMEM((8, 128), jnp.float32),
        pltpu.SemaphoreType.DMA,
        pltpu.SemaphoreType.DMA,
        pltpu.VMEM_SHARED((NPAD, PW), jnp.float32),
    ],
)
def _sc_edge(k_hbm, q_hbm, v_hbm, src_hbm, dst_hbm, bias_hbm, zo_hbm,
             ouna_out, ounb_out, exs_out,
             src_v, dst_v, k_v, q_v, v_v, w_v, b_v, ex_v, sem, semv, oun_sh):
    c = lax.axis_index("c")
    s = lax.axis_index("s")
    wid = c * NS + s
    r0 = s * RPT
    e0 = wid * EPW

    lanes = jnp.arange(16, dtype=jnp.int32)
    zero16 = jnp.zeros((16,), jnp.float32)

    def zero_w_cols(lo, hi):
        for g in range(G):
            rows = lanes + g * 16
            for col in range(lo, hi):
                plsc.store_scatter(w_v, [rows, jnp.full((16,), col, jnp.int32)],
                                   zero16)

    pltpu.sync_copy(zo_hbm.at[pl.ds(r0, RPT)], oun_sh.at[pl.ds(r0, RPT)])
    zero_w_cols(FH + H, PW)
    plsc.subcore_barrier()

    @pl.loop(0, NCHUNK)
    def chunk_a(j):
        base = e0 + j * C
        rr0 = (wid * NCHUNK + j) * 8
        pltpu.sync_copy(src_hbm.at[pl.ds(base, C)], src_v)
        pltpu.sync_copy(dst_hbm.at[pl.ds(base, C)], dst_v)
        pltpu.sync_copy(bias_hbm.at[pl.ds(rr0, 8)], b_v)
        cpk = pltpu.async_copy(k_hbm.at[src_v], k_v, sem)
        cpq = pltpu.async_copy(q_hbm.at[dst_v], q_v, sem)
        cpv = pltpu.async_copy(v_hbm.at[src_v], v_v, semv)
        cpk.wait()
        cpq.wait()

        def score(g, carry2):
            rows = lanes + g * 16
            grow = jnp.full((16,), 0, jnp.int32) + g
            for h in range(H):
                acc = plsc.load_gather(b_v, [grow, lanes * H + h])
                for d in range(HD):
                    col = jnp.full((16,), h * HD + d, jnp.int32)
                    acc = acc + (plsc.load_gather(k_v, [rows, col])
                                 * plsc.load_gather(q_v, [rows, col]))
                exh = jnp.exp(acc)
                plsc.store_scatter(
                    w_v, [rows, jnp.full((16,), FH + h, jnp.int32)], exh)
                plsc.store_scatter(ex_v, [grow, lanes * H + h], exh)
            return carry2

        lax.fori_loop(0, G, score, 0)
        cpv.wait()

        def scale(g, carry2):
            rows = lanes + g * 16
            for h in range(HH):
                exh = plsc.load_gather(
                    w_v, [rows, jnp.full((16,), FH + h, jnp.int32)])
                for d in range(HD):
                    col = jnp.full((16,), h * HD + d, jnp.int32)
                    vv = plsc.load_gather(v_v, [rows, col])
                    plsc.store_scatter(w_v, [rows, col], vv * exh)
            return carry2

        lax.fori_loop(0, G, scale, 0)
        pltpu.sync_copy(ex_v, exs_out.at[pl.ds(rr0, 8)])
        pltpu.sync_copy(w_v, oun_sh.at[dst_v], add=True)

    plsc.subcore_barrier()
    pltpu.sync_copy(oun_sh.at[pl.ds(r0, RPT)], ouna_out.at[c, pl.ds(r0, RPT)])
    plsc.subcore_barrier()

    pltpu.sync_copy(zo_hbm.at[pl.ds(r0, RPT)], oun_sh.at[pl.ds(r0, RPT)])
    zero_w_cols(FH, FH + H)
    plsc.subcore_barrier()

    @pl.loop(0, NCHUNK)
    def chunk_b(j):
        base = e0 + j * C
        rr0 = (wid * NCHUNK + j) * 8
        pltpu.sync_copy(src_hbm.at[pl.ds(base, C)], src_v)
        pltpu.sync_copy(dst_hbm.at[pl.ds(base, C)], dst_v)
        pltpu.sync_copy(exs_out.at[pl.ds(rr0, 8)], ex_v)
        cpv = pltpu.async_copy(v_hbm.at[src_v], v_v, semv)
        cpv.wait()

        def group(g, carry2):
            rows = lanes + g * 16
            grow = jnp.full((16,), 0, jnp.int32) + g
            for h in range(HH, H):
                exh = plsc.load_gather(ex_v, [grow, lanes * H + h])
                for d in range(HD):
                    vcol = jnp.full((16,), h * HD + d, jnp.int32)
                    wcol = jnp.full((16,), (h - HH) * HD + d, jnp.int32)
                    vv = plsc.load_gather(v_v, [rows, vcol])
                    plsc.store_scatter(w_v, [rows, wcol], vv * exh)
            return carry2

        lax.fori_loop(0, G, group, 0)
        pltpu.sync_copy(w_v, oun_sh.at[dst_v], add=True)

    plsc.subcore_barrier()
    pltpu.sync_copy(oun_sh.at[pl.ds(r0, RPT)], ounb_out.at[c, pl.ds(r0, RPT)])


def _final_body(ouna_ref, ounb_ref, sa_ref, sb_ref, wo_ref, out_ref):
    a = ouna_ref[...]
    b = ounb_ref[...]
    asum = a[0] + a[1]
    bsum = b[0] + b[1]
    den = asum[:, FH:FH + H]
    numa = asum[:, :FH]
    numb = bsum[:, :FH]
    dena = jnp.maximum(den @ sa_ref[...], 1e-30)
    denb = jnp.maximum(den @ sb_ref[...], 1e-30)
    wo = wo_ref[...]
    out_ref[...] = (numa / dena) @ wo[:FH, :] + (numb / denb) @ wo[FH:, :]


def _finalize(ouna, ounb, SA, SB, WoT):
    blk = 2048
    return pl.pallas_call(
        _final_body,
        grid=(NPAD // blk,),
        in_specs=[
            pl.BlockSpec((NC, blk, PW), lambda i: (0, i, 0)),
            pl.BlockSpec((NC, blk, PW), lambda i: (0, i, 0)),
            pl.BlockSpec((H, FH), lambda i: (0, 0)),
            pl.BlockSpec((H, FH), lambda i: (0, 0)),
            pl.BlockSpec((FEAT, FEAT), lambda i: (0, 0)),
        ],
        out_specs=pl.BlockSpec((blk, FEAT), lambda i: (i, 0)),
        out_shape=jax.ShapeDtypeStruct((NPAD, FEAT), jnp.float32),
    )(ouna, ounb, SA, SB, WoT)


def kernel(nfeat, edge_index, attn_bias, Wq, Wk, Wv, Wo):
    scaling = HD ** (-0.5)
    q, k, v = _project(nfeat, Wq.T / scaling, Wk.T, Wv.T)
    src = edge_index[0]
    dst = edge_index[1]
    bias_packed = jnp.pad(attn_bias.reshape(E // C, G, 128),
                          ((0, 0), (0, 8 - G), (0, 0))).reshape(ERP, 128)
    zo = jnp.zeros((NPAD, PW), jnp.float32)
    ouna, ounb, _exs = _sc_edge(k, q, v, src, dst, bias_packed, zo)
    SA = (jnp.arange(FH) // HD == jnp.arange(H)[:, None]).astype(jnp.float32)
    SB = (jnp.arange(FH) // HD + HH == jnp.arange(H)[:, None]).astype(jnp.float32)
    out = _finalize(ouna, ounb, SA, SB, Wo.T)
    return out[:N]


# speedup vs baseline: 1.0036x; 1.0036x over previous
import functools

import jax
import jax.numpy as jnp
from jax import lax
from jax.experimental import pallas as pl
from jax.experimental.pallas import tpu as pltpu
from jax.experimental.pallas import tpu_sc as plsc

N = 10000
E = 320000
FEAT = 128
FH = FEAT // 2
H = 8
HD = FEAT // H
HH = H // 2
NC = 2
NS = 16
NW = NC * NS
EPW = E // NW
C = 80
G = C // 16
NCHUNK = EPW // C
NPAD = 10240
RPT = NPAD // NS
PW = 80
CHT = NW * NCHUNK
ERP = CHT * 8


def _proj_body(x_ref, wq_ref, wk_ref, wv_ref, q_ref, k_ref, v_ref):
    x = x_ref[...]
    q_ref[...] = x @ wq_ref[...]
    k_ref[...] = x @ wk_ref[...]
    v_ref[...] = x @ wv_ref[...]


def _project(nfeat, WqT, WkT, WvT):
    blk = 2000
    return pl.pallas_call(
        _proj_body,
        grid=(N // blk,),
        in_specs=[
            pl.BlockSpec((blk, FEAT), lambda i: (i, 0)),
            pl.BlockSpec((FEAT, FEAT), lambda i: (0, 0)),
            pl.BlockSpec((FEAT, FEAT), lambda i: (0, 0)),
            pl.BlockSpec((FEAT, FEAT), lambda i: (0, 0)),
        ],
        out_specs=[
            pl.BlockSpec((blk, FEAT), lambda i: (i, 0)),
            pl.BlockSpec((blk, FEAT), lambda i: (i, 0)),
            pl.BlockSpec((blk, FEAT), lambda i: (i, 0)),
        ],
        out_shape=[jax.ShapeDtypeStruct((N, FEAT), jnp.float32)] * 3,
    )(nfeat, WqT, WkT, WvT)


_MESH = plsc.VectorSubcoreMesh(core_axis_name="c", subcore_axis_name="s")


@functools.partial(
    pl.kernel,
    mesh=_MESH,
    compiler_params=pltpu.CompilerParams(needs_layout_passes=False),
    out_type=[
        jax.ShapeDtypeStruct((NC, NPAD, PW), jnp.float32),
        jax.ShapeDtypeStruct((NC, NPAD, PW), jnp.float32),
        jax.ShapeDtypeStruct((ERP, 128), jnp.float32),
    ],
    scratch_types=[
        pltpu.VMEM((C,), jnp.int32),
        pltpu.VMEM((C,), jnp.int32),
        pltpu.VMEM((C, FEAT), jnp.float32),
        pltpu.VMEM((C, FEAT), jnp.float32),
        pltpu.VMEM((C, FEAT), jnp.float32),
        pltpu.VMEM((C, PW), jnp.float32),
        pltpu.VMEM((8, 128), jnp.float32),
        pltpu.VMEM((8, 128), jnp.float32),
        pltpu.SemaphoreType.DMA,
        pltpu.VMEM_SHARED((NPAD, PW), jnp.float32),
    ],
)
def _sc_edge(k_hbm, q_hbm, v_hbm, src_hbm, dst_hbm, bias_hbm, zo_hbm,
             ouna_out, ounb_out, exs_out,
             src_v, dst_v, k_v, q_v, v_v, w_v, b_v, ex_v, sem, oun_sh):
    c = lax.axis_index("c")
    s = lax.axis_index("s")
    wid = c * NS + s
    r0 = s * RPT
    e0 = wid * EPW

    lanes = jnp.arange(16, dtype=jnp.int32)
    zero16 = jnp.zeros((16,), jnp.float32)

    def zero_w_cols(lo, hi):
        for g in range(G):
            rows = lanes + g * 16
            for col in range(lo, hi):
                plsc.store_scatter(w_v, [rows, jnp.full((16,), col, jnp.int32)],
                                   zero16)

    pltpu.sync_copy(zo_hbm.at[pl.ds(r0, RPT)], oun_sh.at[pl.ds(r0, RPT)])
    zero_w_cols(FH + H, PW)
    plsc.subcore_barrier()

    @pl.loop(0, NCHUNK)
    def chunk_a(j):
        base = e0 + j * C
        rr0 = (wid * NCHUNK + j) * 8
        pltpu.sync_copy(src_hbm.at[pl.ds(base, C)], src_v)
        pltpu.sync_copy(dst_hbm.at[pl.ds(base, C)], dst_v)
        pltpu.sync_copy(bias_hbm.at[pl.ds(rr0, 8)], b_v)
        cpk = pltpu.async_copy(k_hbm.at[src_v], k_v, sem)
        cpq = pltpu.async_copy(q_hbm.at[dst_v], q_v, sem)
        cpv = pltpu.async_copy(v_hbm.at[src_v], v_v, sem)
        cpk.wait()
        cpq.wait()
        cpv.wait()

        def group(g, carry2):
            rows = lanes + g * 16
            grow = jnp.full((16,), 0, jnp.int32) + g
            for h in range(H):
                acc = plsc.load_gather(b_v, [grow, lanes * H + h])
                for d in range(HD):
                    col = jnp.full((16,), h * HD + d, jnp.int32)
                    acc = acc + (plsc.load_gather(k_v, [rows, col])
                                 * plsc.load_gather(q_v, [rows, col]))
                exh = jnp.exp(acc)
                plsc.store_scatter(
                    w_v, [rows, jnp.full((16,), FH + h, jnp.int32)], exh)
                plsc.store_scatter(ex_v, [grow, lanes * H + h], exh)
                if h < HH:
                    for d in range(HD):
                        col = jnp.full((16,), h * HD + d, jnp.int32)
                        vv = plsc.load_gather(v_v, [rows, col])
                        plsc.store_scatter(w_v, [rows, col], vv * exh)
            return carry2

        lax.fori_loop(0, G, group, 0)
        pltpu.sync_copy(ex_v, exs_out.at[pl.ds(rr0, 8)])
        pltpu.sync_copy(w_v, oun_sh.at[dst_v], add=True)

    plsc.subcore_barrier()
    pltpu.sync_copy(oun_sh.at[pl.ds(r0, RPT)], ouna_out.at[c, pl.ds(r0, RPT)])
    plsc.subcore_barrier()

    pltpu.sync_copy(zo_hbm.at[pl.ds(r0, RPT)], oun_sh.at[pl.ds(r0, RPT)])
    zero_w_cols(FH, FH + H)
    plsc.subcore_barrier()

    @pl.loop(0, NCHUNK)
    def chunk_b(j):
        base = e0 + j * C
        rr0 = (wid * NCHUNK + j) * 8
        pltpu.sync_copy(src_hbm.at[pl.ds(base, C)], src_v)
        pltpu.sync_copy(dst_hbm.at[pl.ds(base, C)], dst_v)
        pltpu.sync_copy(exs_out.at[pl.ds(rr0, 8)], ex_v)
        cpv = pltpu.async_copy(v_hbm.at[src_v], v_v, sem)
        cpv.wait()

        def group(g, carry2):
            rows = lanes + g * 16
            grow = jnp.full((16,), 0, jnp.int32) + g
            for h in range(HH, H):
                exh = plsc.load_gather(ex_v, [grow, lanes * H + h])
                for d in range(HD):
                    vcol = jnp.full((16,), h * HD + d, jnp.int32)
                    wcol = jnp.full((16,), (h - HH) * HD + d, jnp.int32)
                    vv = plsc.load_gather(v_v, [rows, vcol])
                    plsc.store_scatter(w_v, [rows, wcol], vv * exh)
            return carry2

        lax.fori_loop(0, G, group, 0)
        pltpu.sync_copy(w_v, oun_sh.at[dst_v], add=True)

    plsc.subcore_barrier()
    pltpu.sync_copy(oun_sh.at[pl.ds(r0, RPT)], ounb_out.at[c, pl.ds(r0, RPT)])


def _final_body(ouna_ref, ounb_ref, sa_ref, sb_ref, wo_ref, out_ref):
    a = ouna_ref[...]
    b = ounb_ref[...]
    asum = a[0] + a[1]
    bsum = b[0] + b[1]
    den = asum[:, FH:FH + H]
    numa = asum[:, :FH]
    numb = bsum[:, :FH]
    dena = jnp.maximum(den @ sa_ref[...], 1e-30)
    denb = jnp.maximum(den @ sb_ref[...], 1e-30)
    wo = wo_ref[...]
    out_ref[...] = (numa / dena) @ wo[:FH, :] + (numb / denb) @ wo[FH:, :]


def _finalize(ouna, ounb, SA, SB, WoT):
    blk = 2048
    return pl.pallas_call(
        _final_body,
        grid=(NPAD // blk,),
        in_specs=[
            pl.BlockSpec((NC, blk, PW), lambda i: (0, i, 0)),
            pl.BlockSpec((NC, blk, PW), lambda i: (0, i, 0)),
            pl.BlockSpec((H, FH), lambda i: (0, 0)),
            pl.BlockSpec((H, FH), lambda i: (0, 0)),
            pl.BlockSpec((FEAT, FEAT), lambda i: (0, 0)),
        ],
        out_specs=pl.BlockSpec((blk, FEAT), lambda i: (i, 0)),
        out_shape=jax.ShapeDtypeStruct((NPAD, FEAT), jnp.float32),
    )(ouna, ounb, SA, SB, WoT)


def kernel(nfeat, edge_index, attn_bias, Wq, Wk, Wv, Wo):
    scaling = HD ** (-0.5)
    q, k, v = _project(nfeat, Wq.T / scaling, Wk.T, Wv.T)
    src = edge_index[0]
    dst = edge_index[1]
    bias_packed = jnp.pad(attn_bias.reshape(E // C, G, 128),
                          ((0, 0), (0, 8 - G), (0, 0))).reshape(ERP, 128)
    zo = jnp.zeros((NPAD, PW), jnp.float32)
    ouna, ounb, _exs = _sc_edge(k, q, v, src, dst, bias_packed, zo)
    SA = (jnp.arange(FH) // HD == jnp.arange(H)[:, None]).astype(jnp.float32)
    SB = (jnp.arange(FH) // HD + HH == jnp.arange(H)[:, None]).astype(jnp.float32)
    out = _finalize(ouna, ounb, SA, SB, Wo.T)
    return out[:N]
